# baseline (speedup 1.0000x reference)
import functools

import jax
import jax.numpy as jnp
from jax import lax
from jax.experimental import pallas as pl
from jax.experimental.pallas import tpu as pltpu
from jax.experimental.pallas import tpu_sc as plsc

EMB = 64
LANES = 16
NUM_WORKERS = 32
BSLICE = 128
NBUF = 7
LOOK_L = 6
LOOK_G = 5


ROWPITCH = 72


def _body(inp_hbm, lab_hbm, tab_hbm, out_hbm, idx_v, rows_v, outb_v,
          rows_p, si, sg, so, *, hist):
    wid = lax.axis_index("s") * 2 + lax.axis_index("c")

    def issue_loads(h, q):
        hs = h // 8
        hr = h % 8 if isinstance(h, int) else lax.rem(h, 8)
        pltpu.async_copy(lab_hbm.at[hs, wid, hr], idx_v[q], si)
        pltpu.async_copy(inp_hbm.at[h, :, wid], outb_v[q], si)

    def wait_loads(h, q):
        hs = h // 8
        hr = h % 8 if isinstance(h, int) else lax.rem(h, 8)
        pltpu.make_async_copy(lab_hbm.at[hs, wid, hr], idx_v[q], si).wait()
        pltpu.make_async_copy(inp_hbm.at[h, :, wid], outb_v[q], si).wait()

    def double_idx(q):
        for g in range(BSLICE // LANES):
            sl = pl.ds(g * LANES, LANES)
            idx_v[q][sl] = lax.shift_left(idx_v[q][sl], 1)

    def issue_gather(q):
        pltpu.async_copy(tab_hbm.at[idx_v[q]], rows_v[q], sg)

    def wait_gather(q):
        pltpu.make_async_copy(tab_hbm.at[idx_v[q]], rows_v[q], sg).wait()

    def issue_out(h, q):
        pltpu.async_copy(outb_v[q], out_hbm.at[h, :, wid], so)

    def wait_out(h, q):
        pltpu.make_async_copy(outb_v[q], out_hbm.at[h, :, wid], so).wait()

    bidx = [lax.iota(jnp.int32, LANES) + g * LANES
            for g in range(BSLICE // LANES)]

    def compute(q):
        rows_q = rows_v[q]
        outb_q = outb_v[q]

        @plsc.parallel_loop(0, BSLICE, unroll=8)
        def _(b):
            for g in range(EMB // LANES):
                sl = pl.ds(g * LANES, LANES)
                rows_p[b, sl] = rows_q[b, sl]

        @plsc.parallel_loop(0, EMB, unroll=4)
        def _(d):
            s = lax.shift_right_logical(d, 3)
            r = lax.bitwise_and(d, 7)
            dcol = jnp.zeros((LANES,), jnp.int32) + d
            for g in range(BSLICE // LANES):
                emb = plsc.load_gather(rows_p, [bidx[g], dcol])
                plsc.addupdate(outb_q.at[s, r, pl.ds(g * LANES, LANES)], emb)

    def _when(cond, fn):
        if isinstance(cond, bool):
            if cond:
                fn()
        else:
            pl.when(cond)(fn)

    def step(h, q):
        def _feed():
            wait_loads(h + LOOK_G, (q + LOOK_G) % NBUF)
            double_idx((q + LOOK_G) % NBUF)
            issue_gather((q + LOOK_G) % NBUF)

        _when(h + LOOK_G < hist, _feed)
        wait_gather(q)
        compute(q)
        issue_out(h, q)
        _when(h >= 1, lambda: wait_out(h - 1, (q - 1) % NBUF))
        _when(h + LOOK_L < hist,
              lambda: issue_loads(h + LOOK_L, (q + LOOK_L) % NBUF))

    for k in range(LOOK_L):
        issue_loads(k, k)
    for k in range(LOOK_G):
        wait_loads(k, k)
        double_idx(k)
        issue_gather(k)

    def multi_step(j, carry):
        h = j * NBUF
        for q in range(NBUF):
            step(h + q, q)
        return carry

    main_steps = (hist // NBUF) * NBUF
    lax.fori_loop(0, hist // NBUF, multi_step, 0)
    for h in range(main_steps, hist):
        step(h, h % NBUF)
    wait_out(hist - 1, (hist - 1) % NBUF)


def kernel(inputs, labels, emb_table):
    batch, hist, emb = inputs.shape
    assert emb == EMB and batch == NUM_WORKERS * BSLICE and hist % 8 == 0
    VOCAB_ROWS = emb_table.shape[0]

    inp5 = jnp.transpose(inputs, (1, 2, 0))
    inp5 = inp5.reshape(hist, 8, EMB // 8, NUM_WORKERS, BSLICE)
    inp5 = jnp.transpose(inp5, (0, 1, 3, 2, 4))

    lab4 = jnp.transpose(labels, (1, 0)).astype(jnp.int32)
    lab4 = lab4.reshape(hist // 8, 8, NUM_WORKERS, BSLICE)
    lab4 = jnp.transpose(lab4, (0, 2, 1, 3))

    mesh = plsc.VectorSubcoreMesh(core_axis_name="c", subcore_axis_name="s")
    run = pl.kernel(
        functools.partial(_body, hist=hist),
        out_type=jax.ShapeDtypeStruct((hist, 8, NUM_WORKERS, EMB // 8, BSLICE),
                                      jnp.float32),
        mesh=mesh,
        scratch_types=(
            [[pltpu.VMEM((BSLICE,), jnp.int32) for _ in range(NBUF)],
             [pltpu.VMEM((BSLICE, EMB), jnp.float32) for _ in range(NBUF)],
             [pltpu.VMEM((EMB // 8, 8, BSLICE), jnp.float32)
              for _ in range(NBUF)],
             pltpu.VMEM((BSLICE, ROWPITCH), jnp.float32)]
            + [pltpu.SemaphoreType.DMA] * 3
        ),
        compiler_params=pltpu.CompilerParams(use_tc_tiling_on_sc=False,
                                             needs_layout_passes=False),
    )
    tab_wide = jnp.pad(emb_table, ((0, 0), (0, EMB)))
    tab2 = tab_wide.reshape(2 * VOCAB_ROWS, EMB)
    out5 = run(inp5, lab4, tab2)
    out = jnp.transpose(out5, (0, 1, 3, 2, 4)).reshape(hist, EMB, batch)
    return jnp.transpose(out, (2, 0, 1))
